# baseline (speedup 1.0000x reference)
import functools

import jax
import jax.numpy as jnp
from jax import lax
from jax.experimental import pallas as pl
from jax.experimental.pallas import tpu as pltpu
from jax.experimental.pallas import tpu_sc as plsc

_NC = 2
_NS = 16
_NW = _NC * _NS
_LPAD = 8


def _merge_body(scale, main_ref, a_ref, bt_ref, out_ref):
    out_ref[...] = main_ref[...] + scale * jnp.dot(
        a_ref[...], bt_ref[...], preferred_element_type=jnp.float32
    )


def _merged_table(main_weight, lora_a, lora_bt, scale):
    v, d = main_weight.shape
    r = lora_a.shape[1]
    block = 4000
    grid = v // block
    return pl.pallas_call(
        functools.partial(_merge_body, scale),
        grid=(grid,),
        in_specs=[
            pl.BlockSpec((block, d), lambda i: (i, 0)),
            pl.BlockSpec((block, r), lambda i: (i, 0)),
            pl.BlockSpec((r, d), lambda i: (0, 0)),
        ],
        out_specs=pl.BlockSpec((block, d), lambda i: (i, 0)),
        out_shape=jax.ShapeDtypeStruct((v, d), jnp.float32),
    )(main_weight, lora_a, lora_bt)


def _make_gather(b, l, lp, d):
    b_per_w = b // _NW
    nchunk = b_per_w // 2
    npairs = nchunk // 2
    chunk = 2 * lp
    n_per_w = nchunk * chunk
    mesh = plsc.VectorSubcoreMesh(
        core_axis_name="c", subcore_axis_name="s", num_cores=_NC, num_subcores=_NS
    )

    @functools.partial(
        pl.kernel,
        out_type=jax.ShapeDtypeStruct((b, l, d), jnp.float32),
        mesh=mesh,
        scratch_types=[
            pltpu.VMEM((n_per_w,), jnp.int32),
            pltpu.VMEM((2, chunk, d), jnp.float32),
            pltpu.SemaphoreType.DMA,
        ],
    )
    def gather(table_hbm, idx_hbm, out_hbm, idx_v, rows_v, gsem):
        wid = lax.axis_index("s") * _NC + lax.axis_index("c")
        bbase = wid * b_per_w
        pltpu.sync_copy(idx_hbm.at[wid], idx_v)

        def fire(j, slot):
            pltpu.async_copy(
                table_hbm.at[idx_v.at[pl.ds(j * chunk, chunk)]],
                rows_v.at[slot],
                gsem,
            )

        def gwait(slot):
            pltpu.make_async_copy(
                table_hbm.at[idx_v.at[pl.ds(0, chunk)]], rows_v.at[slot], gsem
            ).wait()

        def drain(j, slot):
            b0 = bbase + 2 * j
            pltpu.sync_copy(rows_v.at[slot].at[pl.ds(0, l)], out_hbm.at[b0])
            pltpu.sync_copy(rows_v.at[slot].at[pl.ds(lp, l)], out_hbm.at[b0 + 1])

        fire(0, 0)

        def pair(p, carry):
            j0 = 2 * p
            gwait(0)
            fire(j0 + 1, 1)
            drain(j0, 0)
            gwait(1)

            @pl.when(p + 1 < npairs)
            def _():
                fire(j0 + 2, 0)

            drain(j0 + 1, 1)
            return carry

        lax.fori_loop(0, npairs, pair, 0)

    return gather


def kernel(idx, main_weight, lora_A, lora_B):
    b, l = idx.shape
    v, d = main_weight.shape
    rank = lora_A.shape[1]
    alpha = 32.0
    scale = alpha / rank

    merged = _merged_table(main_weight, lora_A, lora_B.T, scale)

    lp = ((l + _LPAD - 1) // _LPAD) * _LPAD
    assert b % (2 * _NW) == 0
    idx_pad = jnp.pad(idx.astype(jnp.int32), ((0, 0), (0, lp - l)))
    idx2 = idx_pad.reshape(_NW, (b // _NW) * lp)
    return _make_gather(b, l, lp, d)(merged, idx2)


# speedup vs baseline: 6.9786x; 6.9786x over previous
import functools

import jax
import jax.numpy as jnp
from jax import lax
from jax.experimental import pallas as pl
from jax.experimental.pallas import tpu as pltpu
from jax.experimental.pallas import tpu_sc as plsc

_NC = 2
_NS = 16
_NW = _NC * _NS
_CHUNK = 128


def _merge_body(scale, main_ref, a_ref, bt_ref, out_ref):
    out_ref[...] = main_ref[...] + scale * jnp.dot(
        a_ref[...], bt_ref[...], preferred_element_type=jnp.float32
    )


def _merged_table(main_weight, lora_a, lora_bt, scale):
    v, d = main_weight.shape
    r = lora_a.shape[1]
    block = 4000
    grid = v // block
    return pl.pallas_call(
        functools.partial(_merge_body, scale),
        grid=(grid,),
        in_specs=[
            pl.BlockSpec((block, d), lambda i: (i, 0)),
            pl.BlockSpec((block, r), lambda i: (i, 0)),
            pl.BlockSpec((r, d), lambda i: (0, 0)),
        ],
        out_specs=pl.BlockSpec((block, d), lambda i: (i, 0)),
        out_shape=jax.ShapeDtypeStruct((v, d), jnp.float32),
    )(main_weight, lora_a, lora_bt)


def _make_gather(nchunk, d):
    n_per_w = nchunk * _CHUNK
    assert nchunk % 2 == 0
    npairs = nchunk // 2
    mesh = plsc.VectorSubcoreMesh(
        core_axis_name="c", subcore_axis_name="s", num_cores=_NC, num_subcores=_NS
    )

    @functools.partial(
        pl.kernel,
        out_type=jax.ShapeDtypeStruct((_NW * n_per_w, d), jnp.float32),
        mesh=mesh,
        scratch_types=[
            pltpu.VMEM((n_per_w,), jnp.int32),
            pltpu.VMEM((2, _CHUNK, d), jnp.float32),
            pltpu.SemaphoreType.DMA,
        ],
    )
    def gather(table_hbm, idx_hbm, out_hbm, idx_v, rows_v, gsem):
        wid = lax.axis_index("s") * _NC + lax.axis_index("c")
        base = wid * n_per_w
        pltpu.sync_copy(idx_hbm.at[wid], idx_v)

        def fire(j, slot):
            pltpu.async_copy(
                table_hbm.at[idx_v.at[pl.ds(j * _CHUNK, _CHUNK)]],
                rows_v.at[slot],
                gsem,
            )

        def gwait(slot):
            pltpu.make_async_copy(
                table_hbm.at[idx_v.at[pl.ds(0, _CHUNK)]], rows_v.at[slot], gsem
            ).wait()

        def drain(j, slot):
            pltpu.sync_copy(rows_v.at[slot], out_hbm.at[pl.ds(base + j * _CHUNK, _CHUNK)])

        fire(0, 0)

        def pair(p, carry):
            j0 = 2 * p
            gwait(0)
            fire(j0 + 1, 1)
            drain(j0, 0)
            gwait(1)

            @pl.when(p + 1 < npairs)
            def _():
                fire(j0 + 2, 0)

            drain(j0 + 1, 1)
            return carry

        lax.fori_loop(0, npairs, pair, 0)

    return gather


def kernel(idx, main_weight, lora_A, lora_B):
    b, l = idx.shape
    v, d = main_weight.shape
    rank = lora_A.shape[1]
    alpha = 32.0
    scale = alpha / rank

    merged = _merged_table(main_weight, lora_A, lora_B.T, scale)

    n = b * l
    assert n % (_NW * _CHUNK) == 0
    nchunk = n // (_NW * _CHUNK)
    idx2 = idx.astype(jnp.int32).T.reshape(_NW, nchunk * _CHUNK)
    rows = _make_gather(nchunk, d)(merged, idx2)
    return rows.reshape(l, b, d).transpose(1, 0, 2)


# speedup vs baseline: 7.8035x; 1.1182x over previous
import functools

import jax
import jax.numpy as jnp
from jax import lax
from jax.experimental import pallas as pl
from jax.experimental.pallas import tpu as pltpu
from jax.experimental.pallas import tpu_sc as plsc

_NC = 2
_NS = 16
_NW = _NC * _NS
_CHUNK = 128


def _merge_body(scale, main_ref, a_ref, bt_ref, out_ref):
    out_ref[...] = main_ref[...] + scale * jnp.dot(
        a_ref[...], bt_ref[...], preferred_element_type=jnp.float32
    )


def _merged_table(main_weight, lora_a, lora_bt, scale):
    v, d = main_weight.shape
    r = lora_a.shape[1]
    block = 4000
    grid = v // block
    return pl.pallas_call(
        functools.partial(_merge_body, scale),
        grid=(grid,),
        in_specs=[
            pl.BlockSpec((block, d), lambda i: (i, 0)),
            pl.BlockSpec((block, r), lambda i: (i, 0)),
            pl.BlockSpec((r, d), lambda i: (0, 0)),
        ],
        out_specs=pl.BlockSpec((block, d), lambda i: (i, 0)),
        out_shape=jax.ShapeDtypeStruct((v, d), jnp.float32),
    )(main_weight, lora_a, lora_bt)


_NBUF = 5


def _make_gather(nchunk, d):
    n_per_w = nchunk * _CHUNK
    assert nchunk % _NBUF == 0
    ngroups = nchunk // _NBUF
    mesh = plsc.VectorSubcoreMesh(
        core_axis_name="c", subcore_axis_name="s", num_cores=_NC, num_subcores=_NS
    )

    @functools.partial(
        pl.kernel,
        out_type=jax.ShapeDtypeStruct((_NW * n_per_w, d), jnp.float32),
        mesh=mesh,
        scratch_types=[
            pltpu.VMEM((n_per_w,), jnp.int32),
            pltpu.VMEM((_NBUF, _CHUNK, d), jnp.float32),
            [pltpu.SemaphoreType.DMA] * _NBUF,
            [pltpu.SemaphoreType.DMA] * _NBUF,
        ],
    )
    def gather(table_hbm, idx_hbm, out_hbm, idx_v, rows_v, gsem, dsem):
        wid = lax.axis_index("s") * _NC + lax.axis_index("c")
        base = wid * n_per_w
        pltpu.sync_copy(idx_hbm.at[wid], idx_v)

        def fire(j, slot):
            pltpu.async_copy(
                table_hbm.at[idx_v.at[pl.ds(j * _CHUNK, _CHUNK)]],
                rows_v.at[slot],
                gsem[slot],
            )

        def gwait(slot):
            pltpu.make_async_copy(
                table_hbm.at[idx_v.at[pl.ds(0, _CHUNK)]], rows_v.at[slot], gsem[slot]
            ).wait()

        def drain(j, slot):
            pltpu.async_copy(
                rows_v.at[slot], out_hbm.at[pl.ds(base + j * _CHUNK, _CHUNK)], dsem[slot]
            )

        def dwait(slot):
            pltpu.make_async_copy(
                rows_v.at[slot], out_hbm.at[pl.ds(base, _CHUNK)], dsem[slot]
            ).wait()

        for k in range(_NBUF - 1):
            fire(k, k)

        def group(g, carry):
            j0 = g * _NBUF
            for k in range(_NBUF):
                s = k
                s3 = (k + _NBUF - 1) % _NBUF
                gwait(s)
                drain(j0 + k, s)
                jnext = j0 + k + _NBUF - 1

                @pl.when(jnext < nchunk)
                def _():
                    @pl.when(g + k > 0)
                    def _():
                        dwait(s3)

                    fire(jnext, s3)

            return carry

        lax.fori_loop(0, ngroups, group, 0)
        for k in range(_NBUF):
            dwait(k)

    return gather


def kernel(idx, main_weight, lora_A, lora_B):
    b, l = idx.shape
    v, d = main_weight.shape
    rank = lora_A.shape[1]
    alpha = 32.0
    scale = alpha / rank

    merged = _merged_table(main_weight, lora_A, lora_B.T, scale)

    n = b * l
    assert n % (_NW * _CHUNK) == 0
    nchunk = n // (_NW * _CHUNK)
    idx2 = idx.astype(jnp.int32).T.reshape(_NW, nchunk * _CHUNK)
    rows = _make_gather(nchunk, d)(merged, idx2)
    return rows.reshape(l, b, d).transpose(1, 0, 2)


# speedup vs baseline: 7.8149x; 1.0015x over previous
import functools

import jax
import jax.numpy as jnp
from jax import lax
from jax.experimental import pallas as pl
from jax.experimental.pallas import tpu as pltpu
from jax.experimental.pallas import tpu_sc as plsc

_NC = 2
_NS = 16
_NW = _NC * _NS
_CHUNK = 128


def _merge_body(scale, main_ref, a_ref, bt_ref, out_ref):
    out_ref[...] = main_ref[...] + scale * jnp.dot(
        a_ref[...], bt_ref[...], preferred_element_type=jnp.float32
    )


def _merged_table(main_weight, lora_a, lora_bt, scale):
    v, d = main_weight.shape
    r = lora_a.shape[1]
    block = 5000
    grid = v // block
    return pl.pallas_call(
        functools.partial(_merge_body, scale),
        grid=(grid,),
        in_specs=[
            pl.BlockSpec((block, d), lambda i: (i, 0)),
            pl.BlockSpec((block, r), lambda i: (i, 0)),
            pl.BlockSpec((r, d), lambda i: (0, 0)),
        ],
        out_specs=pl.BlockSpec((block, d), lambda i: (i, 0)),
        out_shape=jax.ShapeDtypeStruct((v, d), jnp.float32),
    )(main_weight, lora_a, lora_bt)


_NBUF = 5


def _make_gather(nchunk, d):
    n_per_w = nchunk * _CHUNK
    assert nchunk % _NBUF == 0
    ngroups = nchunk // _NBUF
    mesh = plsc.VectorSubcoreMesh(
        core_axis_name="c", subcore_axis_name="s", num_cores=_NC, num_subcores=_NS
    )

    @functools.partial(
        pl.kernel,
        out_type=jax.ShapeDtypeStruct((_NW * n_per_w, d), jnp.float32),
        mesh=mesh,
        scratch_types=[
            pltpu.VMEM((n_per_w,), jnp.int32),
            pltpu.VMEM((_NBUF, _CHUNK, d), jnp.float32),
            [pltpu.SemaphoreType.DMA] * _NBUF,
            [pltpu.SemaphoreType.DMA] * _NBUF,
        ],
    )
    def gather(table_hbm, idx_hbm, out_hbm, idx_v, rows_v, gsem, dsem):
        wid = lax.axis_index("s") * _NC + lax.axis_index("c")
        base = wid * n_per_w
        pltpu.sync_copy(idx_hbm.at[wid], idx_v)

        def fire(j, slot):
            pltpu.async_copy(
                table_hbm.at[idx_v.at[pl.ds(j * _CHUNK, _CHUNK)]],
                rows_v.at[slot],
                gsem[slot],
            )

        def gwait(slot):
            pltpu.make_async_copy(
                table_hbm.at[idx_v.at[pl.ds(0, _CHUNK)]], rows_v.at[slot], gsem[slot]
            ).wait()

        def drain(j, slot):
            pltpu.async_copy(
                rows_v.at[slot], out_hbm.at[pl.ds(base + j * _CHUNK, _CHUNK)], dsem[slot]
            )

        def dwait(slot):
            pltpu.make_async_copy(
                rows_v.at[slot], out_hbm.at[pl.ds(base, _CHUNK)], dsem[slot]
            ).wait()

        for k in range(_NBUF - 1):
            fire(k, k)

        def group(g, carry):
            j0 = g * _NBUF
            for k in range(_NBUF):
                s = k
                s3 = (k + _NBUF - 1) % _NBUF
                gwait(s)
                drain(j0 + k, s)
                jnext = j0 + k + _NBUF - 1

                @pl.when(jnext < nchunk)
                def _():
                    @pl.when(g + k > 0)
                    def _():
                        dwait(s3)

                    fire(jnext, s3)

            return carry

        lax.fori_loop(0, ngroups, group, 0)
        for k in range(_NBUF):
            dwait(k)

    return gather


def kernel(idx, main_weight, lora_A, lora_B):
    b, l = idx.shape
    v, d = main_weight.shape
    rank = lora_A.shape[1]
    alpha = 32.0
    scale = alpha / rank

    merged = _merged_table(main_weight, lora_A, lora_B.T, scale)

    n = b * l
    assert n % (_NW * _CHUNK) == 0
    nchunk = n // (_NW * _CHUNK)
    idx2 = idx.astype(jnp.int32).T.reshape(_NW, nchunk * _CHUNK)
    rows = _make_gather(nchunk, d)(merged, idx2)
    return rows.reshape(l, b, d).transpose(1, 0, 2)
